# baseline (speedup 1.0000x reference)
import functools

import jax
import jax.numpy as jnp
from jax import lax
from jax.experimental import pallas as pl
from jax.experimental.pallas import tpu as pltpu
from jax.experimental.pallas import tpu_sc as plsc

N = 10000
E = 320000
IN_CH = 128
HID = 16
OUT_CH = 64

NC = 2
NS = 16
NW = NC * NS
NPAD = 10240
NP = NPAD // NS

PC = 128
EPW = 10240
EPAD = NW * EPW
PN = EPW // PC
ECH = PN + 8
EWR = ECH * HID

_F32 = jnp.float32
_I32 = jnp.int32


def _zero_rows(ref, n):
    z = jnp.zeros((HID,), _F32)

    @pl.loop(0, n)
    def _(i):
        ref[i] = z


def _make_sc_prop(mode):
    mesh = plsc.VectorSubcoreMesh(core_axis_name="c", subcore_axis_name="s")

    scratch = [
        pltpu.VMEM_SHARED((NPAD, HID), _F32),
        pltpu.VMEM_SHARED((NPAD, HID), _F32),
    ]
    if mode == "first":
        scratch += [pltpu.VMEM_SHARED((NPAD,), _F32)]
    scratch += [
        pltpu.VMEM((PN + 2, PC), _I32),
        pltpu.VMEM((PN, PC), _I32),
        pltpu.VMEM((PC, HID), _F32),
        pltpu.VMEM((PC, HID), _F32),
        pltpu.VMEM((HID, PC), _F32),
        pltpu.VMEM((HID, PC), _F32),
        pltpu.VMEM((PC, HID), _F32),
        pltpu.VMEM((PC, HID), _F32),
        pltpu.VMEM((NP, HID), _F32),
        pltpu.SemaphoreType.DMA,
        pltpu.SemaphoreType.DMA,
        pltpu.SemaphoreType.DMA,
        pltpu.SemaphoreType.DMA,
        pltpu.SemaphoreType.DMA,
        pltpu.SemaphoreType.DMA,
    ]
    if mode == "mid":
        scratch += [
            pltpu.VMEM((NP, HID), _F32),
            pltpu.VMEM((NP, HID), _F32),
            pltpu.VMEM((NP, HID), _F32),
            pltpu.VMEM((1, HID), _F32),
        ]
    else:
        scratch += [
            pltpu.VMEM((PN, PC), _F32),
            pltpu.VMEM((NP,), _F32),
            pltpu.VMEM((NP,), _F32),
            pltpu.VMEM((NP, HID), _F32),
        ]

    def body(refs):
        if mode == "mid":
            (p_h, disb_h, b1_h, src_h, dst_h, ewb_h, agg_out,
             spmem_hp, spmem_agg, vsrc, vdst,
             g0, g1, e0, e1, s0, s1, vnode,
             sg0, sg1, se0, se1, ss0, ss1,
             vp0, vp1, vdis, vb1) = refs
        else:
            (h1_h, ewc_h, src_h, dst_h, ewb_h, agg_out, disb_out,
             spmem_hp, spmem_agg, spmem_deg, vsrc, vdst,
             g0, g1, e0, e1, s0, s1, vnode,
             sg0, sg1, se0, se1, ss0, ss1,
             vew, vdeg, vdis1, vdisb) = refs

        c = lax.axis_index("c")
        s = lax.axis_index("s")
        row0 = s * NP

        if mode == "mid":
            pltpu.sync_copy(p_h.at[0, pl.ds(row0, NP)], vp0)
            pltpu.sync_copy(p_h.at[1, pl.ds(row0, NP)], vp1)
            pltpu.sync_copy(disb_h.at[pl.ds(row0, NP)], vdis)
            pltpu.sync_copy(b1_h, vb1)
            bv = vb1[0]

            @pl.loop(0, NP)
            def _(i):
                d = vdis[i]
                vnode[i] = d * jnp.maximum(
                    d * (vp0[i] + vp1[i]) + bv, 0.0)
        else:
            z16 = jnp.zeros((16,), _F32)

            @pl.loop(0, NP // 16)
            def _(kk):
                vdeg[pl.ds(16 * kk, 16)] = z16

            pltpu.sync_copy(vdeg, spmem_deg.at[pl.ds(row0, NP)])
            plsc.subcore_barrier()

            mine = c * NS + s
            theirs = (1 - c) * NS + s
            for half_idx in range(2):
                wrow = theirs if half_idx == 0 else mine
                pltpu.sync_copy(dst_h.at[wrow, pl.ds(0, PN)], vdst)
                pltpu.sync_copy(ewc_h.at[wrow, pl.ds(0, PN)], vew)

                @pl.loop(0, PN)
                def _(j):
                    pltpu.sync_copy(vew.at[j], spmem_deg.at[vdst.at[j]],
                                    add=True)

            plsc.subcore_barrier()
            pltpu.sync_copy(spmem_deg.at[pl.ds(row0, NP)], vdeg)

            pltpu.sync_copy(h1_h.at[pl.ds(row0, NP)], vnode)

            magic = jnp.full((16,), 0x5F3759DF, _I32)
            half = jnp.full((16,), 0.5, _F32)
            th = jnp.full((16,), 1.5, _F32)
            one = jnp.full((16,), 1.0, _F32)

            @pl.loop(0, NP // 16)
            def _(kk):
                sl = pl.ds(16 * kk, 16)
                d = vdeg[sl] + one
                y = lax.bitcast_convert_type(
                    magic - lax.shift_right_logical(
                        lax.bitcast_convert_type(d, _I32), 1), _F32)
                hd = half * d
                for _ in range(3):
                    y = y * (th - hd * y * y)
                vdis1[sl] = y

            @pl.loop(0, NP)
            def _(i):
                dv = plsc.load_gather(vdis1, [jnp.full((16,), i, _I32)])
                vdisb[i] = dv
                vnode[i] = vnode[i] * dv

            @pl.when(c == 0)
            def _():
                pltpu.sync_copy(vdisb, disb_out.at[pl.ds(row0, NP)])

        pltpu.sync_copy(vnode, spmem_hp.at[pl.ds(row0, NP)])

        @pl.when(c == 0)
        def _():
            pltpu.sync_copy(vnode, spmem_agg.at[pl.ds(row0, NP)])

        @pl.when(c == 1)
        def _():
            _zero_rows(vnode, NP)
            pltpu.sync_copy(vnode, spmem_agg.at[pl.ds(row0, NP)])

        plsc.subcore_barrier()

        w = c * NS + s
        pltpu.sync_copy(src_h.at[w, pl.ds(0, PN + 2)], vsrc)
        if mode == "mid":
            pltpu.sync_copy(dst_h.at[w, pl.ds(0, PN)], vdst)

        bufs = ((g0, e0, s0, sg0, se0, ss0), (g1, e1, s1, sg1, se1, ss1))

        def scale(gb, eb, sb):
            @pl.loop(0, PC // 8)
            def _(i):
                for u in range(8):
                    r = 8 * i + u
                    sb[r] = gb[r] * eb[i, pl.ds(16 * u, 16)]

        def prefetch(b, j):
            gb, eb, _, sg, se, _ = bufs[b]
            pltpu.async_copy(spmem_hp.at[vsrc.at[j]], gb, sg)
            pltpu.async_copy(ewb_h.at[w, pl.ds(HID * j, HID)], eb, se)

        prefetch(0, 0)
        prefetch(1, 1)

        def wait_eb(eb, se):
            pltpu.make_async_copy(
                ewb_h.at[w, pl.ds(0, HID)], eb, se).wait()

        for b in range(2):
            gb, eb, sb, sg, se, ss = bufs[b]
            pltpu.make_async_copy(spmem_hp.at[vsrc.at[b]], gb, sg).wait()
            wait_eb(eb, se)
            scale(gb, eb, sb)
            pltpu.async_copy(sb, spmem_agg.at[vdst.at[b]], ss, add=True)
            prefetch(b, b + 2)

        @pl.loop(1, PN // 2)
        def _(p):
            for b in range(2):
                gb, eb, sb, sg, se, ss = bufs[b]
                j = 2 * p + b
                pltpu.make_async_copy(
                    spmem_hp.at[vsrc.at[j]], gb, sg).wait()
                wait_eb(eb, se)
                pltpu.make_async_copy(
                    sb, spmem_agg.at[vdst.at[j]], ss).wait()
                scale(gb, eb, sb)
                pltpu.async_copy(sb, spmem_agg.at[vdst.at[j]], ss, add=True)
                prefetch(b, j + 2)

        for b in range(2):
            gb, eb, sb, sg, se, ss = bufs[b]
            pltpu.make_async_copy(
                sb, spmem_agg.at[vdst.at[PN - 2 + b]], ss).wait()
            pltpu.make_async_copy(
                spmem_hp.at[vsrc.at[PN + b]], gb, sg).wait()
            wait_eb(eb, se)

        plsc.subcore_barrier()

        pltpu.sync_copy(spmem_agg.at[pl.ds(row0, NP)], vnode)
        pltpu.sync_copy(vnode, agg_out.at[c, pl.ds(row0, NP)])

    if mode == "mid":
        def k(p_h, disb_h, b1_h, src_h, dst_h, ew_h, agg_out, *scr):
            body((p_h, disb_h, b1_h, src_h, dst_h, ew_h, agg_out) + scr)

        out_type = jax.ShapeDtypeStruct((NC, NPAD, HID), _F32)
    else:
        def k(h1_h, ewc_h, src_h, dst_h, ew_h, agg_out, disb_out, *scr):
            body((h1_h, ewc_h, src_h, dst_h, ew_h, agg_out, disb_out)
                 + scr)

        out_type = (jax.ShapeDtypeStruct((NC, NPAD, HID), _F32),
                    jax.ShapeDtypeStruct((NPAD, HID), _F32))

    return functools.partial(
        pl.kernel,
        out_type=out_type,
        mesh=mesh,
        compiler_params=pltpu.CompilerParams(needs_layout_passes=False, use_tc_tiling_on_sc=False),
        scratch_types=scratch,
    )(k)


def _tc_first(xp, W1, ew_p, krep):
    nb = NPAD // NW

    def body(x_ref, w_ref, f_ref, k_ref, h1_ref, ewb_ref):
        h1_ref[...] = jnp.dot(x_ref[...], w_ref[...],
                              preferred_element_type=_F32)
        m = jnp.dot(f_ref[0], k_ref[...], preferred_element_type=_F32)
        ewb_ref[0] = m.reshape(EWR, 128)

    h1, ewb = pl.pallas_call(
        body,
        grid=(NW,),
        in_specs=[
            pl.BlockSpec((nb, IN_CH), lambda i: (i, 0)),
            pl.BlockSpec((IN_CH, HID), lambda i: (0, 0)),
            pl.BlockSpec((1, ECH, PC), lambda i: (i, 0, 0)),
            pl.BlockSpec((PC, PC * HID), lambda i: (0, 0)),
        ],
        out_specs=(
            pl.BlockSpec((nb, HID), lambda i: (i, 0)),
            pl.BlockSpec((1, EWR, 128), lambda i: (i, 0, 0)),
        ),
        out_shape=(
            jax.ShapeDtypeStruct((NPAD, HID), _F32),
            jax.ShapeDtypeStruct((NW, EWR, 128), _F32),
        ),
    )(xp, W1, ew_p, krep)
    return h1, ewb


def _tc_fin(q, disb, W2, b2_row):
    def body(q0_ref, q1_ref, disb_ref, w_ref, b_ref, o_ref):
        t = (q0_ref[...] + q1_ref[...]) * disb_ref[...]
        sv = jnp.dot(t, w_ref[...], preferred_element_type=_F32) + b_ref[...]
        m = jnp.max(sv, axis=1, keepdims=True)
        lse = jnp.log(jnp.sum(jnp.exp(sv - m), axis=1, keepdims=True)) + m
        o_ref[...] = sv - lse

    return pl.pallas_call(
        body,
        out_shape=jax.ShapeDtypeStruct((NPAD, OUT_CH), _F32),
    )(q[0], q[1], disb, W2, b2_row)


def kernel(x, edge_index, edge_weight, W1, b1, W2, b2):
    src = edge_index[0].astype(_I32)
    dst = edge_index[1].astype(_I32)
    ew = edge_weight.astype(_F32)

    zc = jnp.zeros((NW, ECH - PN, PC), _I32)

    def to_chunks(a):
        a = jnp.pad(a, (0, EPAD - E)).reshape(NW, PN, PC)
        return jnp.concatenate([a, zc.astype(a.dtype)], axis=1)

    src_p = to_chunks(src)
    dst_p = to_chunks(dst)
    ew_p = to_chunks(ew)
    krep = jnp.repeat(jnp.eye(PC, dtype=_F32), HID, axis=1)
    xp = jnp.pad(x, ((0, NPAD - N), (0, 0)))

    h1, ewb = _tc_first(xp, W1, ew_p, krep)
    p, disb = _make_sc_prop("first")(h1, ew_p, src_p, dst_p, ewb)
    q = _make_sc_prop("mid")(p, disb, b1.reshape(1, HID), src_p, dst_p, ewb)
    out = _tc_fin(q, disb, W2, b2.reshape(1, OUT_CH))
    return out[:N]


# speedup vs baseline: 1.0944x; 1.0944x over previous
import functools

import jax
import jax.numpy as jnp
from jax import lax
from jax.experimental import pallas as pl
from jax.experimental.pallas import tpu as pltpu
from jax.experimental.pallas import tpu_sc as plsc

N = 10000
E = 320000
IN_CH = 128
HID = 16
OUT_CH = 64

NC = 2
NS = 16
NW = NC * NS
NPAD = 10240
NP = NPAD // NS

PC = 128
EPW = 10240
EPAD = NW * EPW
PN = EPW // PC
ECH = PN + 8
EWR = ECH * HID

_F32 = jnp.float32
_I32 = jnp.int32


def _zero_rows(ref, n):
    z = jnp.zeros((HID,), _F32)

    @pl.loop(0, n)
    def _(i):
        ref[i] = z


def _sc_deg(dst_p, ew_p):
    mesh = plsc.VectorSubcoreMesh(core_axis_name="c", subcore_axis_name="s")

    @functools.partial(
        pl.kernel,
        out_type=jax.ShapeDtypeStruct((NC, NPAD), _F32),
        mesh=mesh,
        compiler_params=pltpu.CompilerParams(needs_layout_passes=False, use_tc_tiling_on_sc=False),
        scratch_types=[
            pltpu.VMEM_SHARED((NPAD,), _F32),
            pltpu.VMEM((PN, PC), _I32),
            pltpu.VMEM((PN, PC), _F32),
            pltpu.VMEM((NP,), _F32),
        ],
    )
    def k(dst_h, ew_h, deg_out, spmem_deg, vdst, vew, vslice):
        c = lax.axis_index("c")
        s = lax.axis_index("s")
        row0 = s * NP

        z16 = jnp.zeros((16,), _F32)
        for kk in range(NP // 16):
            vslice[pl.ds(16 * kk, 16)] = z16
        pltpu.sync_copy(vslice, spmem_deg.at[pl.ds(row0, NP)])
        plsc.subcore_barrier()

        w = c * NS + s
        pltpu.sync_copy(dst_h.at[w, pl.ds(0, PN)], vdst)
        pltpu.sync_copy(ew_h.at[w, pl.ds(0, PN)], vew)

        @pl.loop(0, PN)
        def _(j):
            pltpu.sync_copy(vew.at[j], spmem_deg.at[vdst.at[j]], add=True)

        plsc.subcore_barrier()

        pltpu.sync_copy(spmem_deg.at[pl.ds(row0, NP)], vslice)
        pltpu.sync_copy(vslice, deg_out.at[c, pl.ds(row0, NP)])

    return k(dst_p, ew_p)


def _make_sc_prop(mode):
    mesh = plsc.VectorSubcoreMesh(core_axis_name="c", subcore_axis_name="s")

    scratch = [
        pltpu.VMEM_SHARED((NPAD, HID), _F32),
        pltpu.VMEM_SHARED((NPAD, HID), _F32),
        pltpu.VMEM((PN + 2, PC), _I32),
        pltpu.VMEM((PN, PC), _I32),
        pltpu.VMEM((PC, HID), _F32),
        pltpu.VMEM((PC, HID), _F32),
        pltpu.VMEM((HID, PC), _F32),
        pltpu.VMEM((HID, PC), _F32),
        pltpu.VMEM((PC, HID), _F32),
        pltpu.VMEM((PC, HID), _F32),
        pltpu.VMEM((NP, HID), _F32),
        pltpu.SemaphoreType.DMA,
        pltpu.SemaphoreType.DMA,
        pltpu.SemaphoreType.DMA,
        pltpu.SemaphoreType.DMA,
        pltpu.SemaphoreType.DMA,
        pltpu.SemaphoreType.DMA,
    ]
    if mode == "mid":
        scratch += [
            pltpu.VMEM((NP, HID), _F32),
            pltpu.VMEM((NP, HID), _F32),
            pltpu.VMEM((NP, HID), _F32),
            pltpu.VMEM((1, HID), _F32),
        ]
    else:
        scratch += [
            pltpu.VMEM((NP,), _F32),
            pltpu.VMEM((NP,), _F32),
            pltpu.VMEM((NP,), _F32),
            pltpu.VMEM((NP, HID), _F32),
        ]

    def body(refs):
        if mode == "mid":
            (p_h, disb_h, b1_h, src_h, dst_h, ewb_h, agg_out,
             spmem_hp, spmem_agg, vsrc, vdst,
             g0, g1, e0, e1, s0, s1, vnode,
             sg0, sg1, se0, se1, ss0, ss1,
             vp0, vp1, vdis, vb1) = refs
        else:
            (h1_h, deg_h, src_h, dst_h, ewb_h, agg_out, disb_out,
             spmem_hp, spmem_agg, vsrc, vdst,
             g0, g1, e0, e1, s0, s1, vnode,
             sg0, sg1, se0, se1, ss0, ss1,
             vdeg0, vdeg1, vdis1, vdisb) = refs

        c = lax.axis_index("c")
        s = lax.axis_index("s")
        row0 = s * NP

        if mode == "mid":
            pltpu.sync_copy(p_h.at[0, pl.ds(row0, NP)], vp0)
            pltpu.sync_copy(p_h.at[1, pl.ds(row0, NP)], vp1)
            pltpu.sync_copy(disb_h.at[pl.ds(row0, NP)], vdis)
            pltpu.sync_copy(b1_h, vb1)
            bv = vb1[0]

            @pl.loop(0, NP)
            def _(i):
                d = vdis[i]
                vnode[i] = d * jnp.maximum(
                    d * (vp0[i] + vp1[i]) + bv, 0.0)
        else:
            pltpu.sync_copy(h1_h.at[pl.ds(row0, NP)], vnode)
            pltpu.sync_copy(deg_h.at[0, pl.ds(row0, NP)], vdeg0)
            pltpu.sync_copy(deg_h.at[1, pl.ds(row0, NP)], vdeg1)

            magic = jnp.full((16,), 0x5F3759DF, _I32)
            half = jnp.full((16,), 0.5, _F32)
            th = jnp.full((16,), 1.5, _F32)
            one = jnp.full((16,), 1.0, _F32)

            @pl.loop(0, NP // 16)
            def _(kk):
                sl = pl.ds(16 * kk, 16)
                d = vdeg0[sl] + vdeg1[sl] + one
                y = lax.bitcast_convert_type(
                    magic - lax.shift_right_logical(
                        lax.bitcast_convert_type(d, _I32), 1), _F32)
                hd = half * d
                for _ in range(3):
                    y = y * (th - hd * y * y)
                vdis1[sl] = y

            @pl.loop(0, NP)
            def _(i):
                dv = plsc.load_gather(vdis1, [jnp.full((16,), i, _I32)])
                vdisb[i] = dv
                vnode[i] = vnode[i] * dv

            @pl.when(c == 0)
            def _():
                pltpu.sync_copy(vdisb, disb_out.at[pl.ds(row0, NP)])

        pltpu.sync_copy(vnode, spmem_hp.at[pl.ds(row0, NP)])

        @pl.when(c == 0)
        def _():
            pltpu.sync_copy(vnode, spmem_agg.at[pl.ds(row0, NP)])

        @pl.when(c == 1)
        def _():
            _zero_rows(vnode, NP)
            pltpu.sync_copy(vnode, spmem_agg.at[pl.ds(row0, NP)])

        plsc.subcore_barrier()

        w = c * NS + s
        pltpu.sync_copy(src_h.at[w, pl.ds(0, PN + 2)], vsrc)
        pltpu.sync_copy(dst_h.at[w, pl.ds(0, PN)], vdst)

        bufs = ((g0, e0, s0, sg0, se0, ss0), (g1, e1, s1, sg1, se1, ss1))

        def scale(gb, eb, sb):
            @pl.loop(0, PC // 8)
            def _(i):
                for u in range(8):
                    r = 8 * i + u
                    sb[r] = gb[r] * eb[i, pl.ds(16 * u, 16)]

        def prefetch(b, j):
            gb, eb, _, sg, se, _ = bufs[b]
            pltpu.async_copy(spmem_hp.at[vsrc.at[j]], gb, sg)
            pltpu.async_copy(ewb_h.at[w, pl.ds(HID * j, HID)], eb, se)

        prefetch(0, 0)
        prefetch(1, 1)

        def wait_eb(eb, se):
            pltpu.make_async_copy(
                ewb_h.at[w, pl.ds(0, HID)], eb, se).wait()

        for b in range(2):
            gb, eb, sb, sg, se, ss = bufs[b]
            pltpu.make_async_copy(spmem_hp.at[vsrc.at[b]], gb, sg).wait()
            wait_eb(eb, se)
            scale(gb, eb, sb)
            pltpu.async_copy(sb, spmem_agg.at[vdst.at[b]], ss, add=True)
            prefetch(b, b + 2)

        @pl.loop(1, PN // 2)
        def _(p):
            for b in range(2):
                gb, eb, sb, sg, se, ss = bufs[b]
                j = 2 * p + b
                pltpu.make_async_copy(
                    spmem_hp.at[vsrc.at[j]], gb, sg).wait()
                wait_eb(eb, se)
                pltpu.make_async_copy(
                    sb, spmem_agg.at[vdst.at[j]], ss).wait()
                scale(gb, eb, sb)
                pltpu.async_copy(sb, spmem_agg.at[vdst.at[j]], ss, add=True)
                prefetch(b, j + 2)

        for b in range(2):
            gb, eb, sb, sg, se, ss = bufs[b]
            pltpu.make_async_copy(
                sb, spmem_agg.at[vdst.at[PN - 2 + b]], ss).wait()
            pltpu.make_async_copy(
                spmem_hp.at[vsrc.at[PN + b]], gb, sg).wait()
            wait_eb(eb, se)

        plsc.subcore_barrier()

        pltpu.sync_copy(spmem_agg.at[pl.ds(row0, NP)], vnode)
        pltpu.sync_copy(vnode, agg_out.at[c, pl.ds(row0, NP)])

    if mode == "mid":
        def k(p_h, disb_h, b1_h, src_h, dst_h, ew_h, agg_out, *scr):
            body((p_h, disb_h, b1_h, src_h, dst_h, ew_h, agg_out) + scr)

        out_type = jax.ShapeDtypeStruct((NC, NPAD, HID), _F32)
    else:
        def k(h1_h, deg_h, src_h, dst_h, ew_h, agg_out, disb_out, *scr):
            body((h1_h, deg_h, src_h, dst_h, ew_h, agg_out, disb_out)
                 + scr)

        out_type = (jax.ShapeDtypeStruct((NC, NPAD, HID), _F32),
                    jax.ShapeDtypeStruct((NPAD, HID), _F32))

    return functools.partial(
        pl.kernel,
        out_type=out_type,
        mesh=mesh,
        compiler_params=pltpu.CompilerParams(needs_layout_passes=False, use_tc_tiling_on_sc=False),
        scratch_types=scratch,
    )(k)


def _tc_first(xp, W1, ew_p, krep):
    nb = NPAD // NW

    def body(x_ref, w_ref, f_ref, k_ref, h1_ref, ewb_ref):
        h1_ref[...] = jnp.dot(x_ref[...], w_ref[...],
                              preferred_element_type=_F32)
        m = jnp.dot(f_ref[0], k_ref[...], preferred_element_type=_F32)
        ewb_ref[0] = m.reshape(EWR, 128)

    h1, ewb = pl.pallas_call(
        body,
        grid=(NW,),
        in_specs=[
            pl.BlockSpec((nb, IN_CH), lambda i: (i, 0)),
            pl.BlockSpec((IN_CH, HID), lambda i: (0, 0)),
            pl.BlockSpec((1, ECH, PC), lambda i: (i, 0, 0)),
            pl.BlockSpec((PC, PC * HID), lambda i: (0, 0)),
        ],
        out_specs=(
            pl.BlockSpec((nb, HID), lambda i: (i, 0)),
            pl.BlockSpec((1, EWR, 128), lambda i: (i, 0, 0)),
        ),
        out_shape=(
            jax.ShapeDtypeStruct((NPAD, HID), _F32),
            jax.ShapeDtypeStruct((NW, EWR, 128), _F32),
        ),
    )(xp, W1, ew_p, krep)
    return h1, ewb


def _tc_fin(q, disb, W2, b2_row):
    def body(q0_ref, q1_ref, disb_ref, w_ref, b_ref, o_ref):
        t = (q0_ref[...] + q1_ref[...]) * disb_ref[...]
        sv = jnp.dot(t, w_ref[...], preferred_element_type=_F32) + b_ref[...]
        m = jnp.max(sv, axis=1, keepdims=True)
        lse = jnp.log(jnp.sum(jnp.exp(sv - m), axis=1, keepdims=True)) + m
        o_ref[...] = sv - lse

    return pl.pallas_call(
        body,
        out_shape=jax.ShapeDtypeStruct((NPAD, OUT_CH), _F32),
    )(q[0], q[1], disb, W2, b2_row)


def kernel(x, edge_index, edge_weight, W1, b1, W2, b2):
    src = edge_index[0].astype(_I32)
    dst = edge_index[1].astype(_I32)
    ew = edge_weight.astype(_F32)

    zc = jnp.zeros((NW, ECH - PN, PC), _I32)

    def to_chunks(a):
        a = jnp.pad(a, (0, EPAD - E)).reshape(NW, PN, PC)
        return jnp.concatenate([a, zc.astype(a.dtype)], axis=1)

    src_p = to_chunks(src)
    dst_p = to_chunks(dst)
    ew_p = to_chunks(ew)
    krep = jnp.repeat(jnp.eye(PC, dtype=_F32), HID, axis=1)
    xp = jnp.pad(x, ((0, NPAD - N), (0, 0)))

    h1, ewb = _tc_first(xp, W1, ew_p, krep)
    degp = _sc_deg(dst_p, ew_p)
    p, disb = _make_sc_prop("first")(h1, degp, src_p, dst_p, ewb)
    q = _make_sc_prop("mid")(p, disb, b1.reshape(1, HID), src_p, dst_p, ewb)
    out = _tc_fin(q, disb, W2, b2.reshape(1, OUT_CH))
    return out[:N]
